# baseline (speedup 1.0000x reference)
import functools

import jax
import jax.numpy as jnp
from jax import lax
from jax.experimental import pallas as pl
from jax.experimental.pallas import tpu as pltpu
from jax.experimental.pallas import tpu_sc as plsc

V = 100000
NF = 26
D = 32
B = 16384
EM = NF * D
H1 = 128
H2 = 64
NW = 32
CPW = EM // NW
OC = 4096
EPS = 1e-5
BT = 1024
T = B // BT


NB = 6
BSZ = 16640
LBSZ = V - 5 * BSZ
SIZES = [BSZ] * 5 + [LBSZ]
BASES = [j * BSZ for j in range(NB)]


def _sc_gather_t(xtf, tabT):
    mesh = plsc.VectorSubcoreMesh(core_axis_name="c", subcore_axis_name="s")
    lanes16 = lambda: lax.iota(jnp.int32, 16)

    @functools.partial(
        pl.kernel,
        mesh=mesh,
        out_type=jax.ShapeDtypeStruct((EM, B), jnp.float32),
        scratch_types=[
            pltpu.VMEM((B,), jnp.float32),
            pltpu.VMEM((B + 16,), jnp.int32),
            pltpu.VMEM((B + 16,), jnp.int32),
            pltpu.VMEM((LBSZ,), jnp.float32),
            pltpu.VMEM((LBSZ,), jnp.float32),
            pltpu.VMEM((B,), jnp.float32),
            pltpu.VMEM((B,), jnp.float32),
            pltpu.SMEM((16,), jnp.int32),
            pltpu.SemaphoreType.DMA,
            pltpu.SemaphoreType.DMA,
            pltpu.SemaphoreType.DMA,
            pltpu.SemaphoreType.DMA,
        ],
        compiler_params=pltpu.CompilerParams(use_tc_tiling_on_sc=True,
                                             needs_layout_passes=False),
    )
    def gk(x_hbm, tab_hbm, out_hbm, xraw, bidx, bpos, cbufa, cbufb,
           orowa, orowb, meta, psema, psemb, osema, osemb):
        wid = lax.axis_index("s") * 2 + lax.axis_index("c")
        c0 = wid * CPW
        cbufs = [cbufa, cbufb]
        psems = [psema, psemb]
        orows = [orowa, orowb]
        osems = [osema, osemb]

        def fire_bin(c, j):
            pltpu.make_async_copy(
                tab_hbm.at[c, pl.ds(BASES[j], SIZES[j])],
                cbufs[j % 2].at[pl.ds(0, SIZES[j])],
                psems[j % 2]).start()

        def wait_bin(c, j):
            pltpu.make_async_copy(
                tab_hbm.at[c, pl.ds(BASES[j], SIZES[j])],
                cbufs[j % 2].at[pl.ds(0, SIZES[j])],
                psems[j % 2]).wait()

        def partition(f):
            pltpu.sync_copy(x_hbm.at[f], xraw)
            off = 0
            for b in range(NB):
                def scan(k, o):
                    iv = plsc.bitcast(xraw[pl.ds(k * 16, 16)], jnp.int32)
                    m = jnp.logical_and(iv >= BASES[b], iv < BASES[b] + SIZES[b])
                    plsc.store_compressed(bidx.at[pl.ds(o, 16)], iv - BASES[b],
                                          mask=m)
                    plsc.store_compressed(bpos.at[pl.ds(o, 16)],
                                          k * 16 + lanes16(), mask=m)
                    return o + jnp.sum(m.astype(jnp.int32))

                end = lax.fori_loop(0, B // 16, scan, off)
                meta[2 * b] = off
                meta[2 * b + 1] = end - off
                off = end

        def gather_bin(j, orow):
            start = meta[2 * j]
            n = meta[2 * j + 1]

            def vec(k, carry):
                o = start + k * 16
                iv = bidx[pl.ds(o, 16)]
                pv = bpos[pl.ds(o, 16)]
                msk = (k * 16 + lanes16()) < n
                ivc = jnp.minimum(jnp.maximum(iv, 0), SIZES[j] - 1)
                vals = plsc.load_gather(cbufs[j % 2], [ivc], mask=msk)
                plsc.store_scatter(orow, [pv], vals, mask=msk)
                return carry

            lax.fori_loop(0, (n + 15) // 16, vec, 0)

        fire_bin(c0, 0)
        fire_bin(c0, 1)

        def pair(i, carry):
            for half in range(2):
                ch = 2 * i + half
                c = c0 + ch
                f = c // D
                orow = orows[half]

                @pl.when(jnp.logical_or(ch == 0, f != (c - 1) // D))
                def _():
                    partition(f)

                @pl.when(i >= 1)
                def _():
                    pltpu.make_async_copy(orow, out_hbm.at[c], osems[half]).wait()

                for j in range(NB):
                    wait_bin(c, j)
                    gather_bin(j, orow)
                    nj = j + 2
                    if nj < NB:
                        fire_bin(c, nj)
                    else:
                        @pl.when(ch + 1 < CPW)
                        def _():
                            fire_bin(c + 1, nj - NB)

                pltpu.make_async_copy(orow, out_hbm.at[c], osems[half]).start()
            return carry

        lax.fori_loop(0, CPW // 2, pair, 0)
        pltpu.make_async_copy(orowa, out_hbm.at[c0 + CPW - 2], osema).wait()
        pltpu.make_async_copy(orowb, out_hbm.at[c0 + CPW - 1], osemb).wait()

    return gk(xtf, tabT)


def _mlp_body(emb_ref, w1_ref, b1_ref, g1_ref, be1_ref, w2_ref, b2_ref,
              g2_ref, be2_ref, w3_ref, b3_ref, out_ref,
              h1_ref, h2_ref, acc1_ref, acc2_ref):
    p = pl.program_id(0)
    t = pl.program_id(1)

    @pl.when(jnp.logical_and(p == 0, t == 0))
    def _():
        acc1_ref[...] = jnp.zeros_like(acc1_ref)
        acc2_ref[...] = jnp.zeros_like(acc2_ref)

    @pl.when(p == 0)
    def _():
        h = jnp.dot(w1_ref[...], emb_ref[...], preferred_element_type=jnp.float32)
        h = jnp.maximum(h + b1_ref[...], 0.0)
        h1_ref[:, pl.ds(t * BT, BT)] = h
        acc1_ref[:, 0:1] += jnp.sum(h, axis=1, keepdims=True)
        acc1_ref[:, 1:2] += jnp.sum(h * h, axis=1, keepdims=True)

    @pl.when(jnp.logical_and(p == 1, t == 0))
    def _():
        mean = acc1_ref[:, 0:1] * (1.0 / B)
        var = acc1_ref[:, 1:2] * (1.0 / B) - mean * mean
        sc = g1_ref[...] * lax.rsqrt(var + EPS)
        acc1_ref[:, 2:3] = sc
        acc1_ref[:, 3:4] = be1_ref[...] - mean * sc

    @pl.when(p == 1)
    def _():
        h1n = h1_ref[:, pl.ds(t * BT, BT)] * acc1_ref[:, 2:3] + acc1_ref[:, 3:4]
        h = jnp.dot(w2_ref[...], h1n, preferred_element_type=jnp.float32)
        h = jnp.maximum(h + b2_ref[...], 0.0)
        h2_ref[:, pl.ds(t * BT, BT)] = h
        acc2_ref[:, 0:1] += jnp.sum(h, axis=1, keepdims=True)
        acc2_ref[:, 1:2] += jnp.sum(h * h, axis=1, keepdims=True)

    @pl.when(jnp.logical_and(p == 2, t == 0))
    def _():
        mean = acc2_ref[:, 0:1] * (1.0 / B)
        var = acc2_ref[:, 1:2] * (1.0 / B) - mean * mean
        sc = g2_ref[...] * lax.rsqrt(var + EPS)
        acc2_ref[:, 2:3] = sc
        acc2_ref[:, 3:4] = be2_ref[...] - mean * sc

    @pl.when(p == 2)
    def _():
        h2n = h2_ref[:, pl.ds(t * BT, BT)] * acc2_ref[:, 2:3] + acc2_ref[:, 3:4]
        o = jnp.sum(h2n * w3_ref[...], axis=0, keepdims=True) + b3_ref[...]
        out_ref[...] = o


def _mlp_t(embT, W1T, b1, g1, be1, W2T, b2, g2, be2, w3, b3, interpret=False):
    full = lambda shape: pl.BlockSpec(shape, lambda p, t: (0, 0))
    return pl.pallas_call(
        _mlp_body,
        grid=(3, T),
        in_specs=[
            pl.BlockSpec((EM, BT), lambda p, t: (0, jnp.where(p == 0, t, 0))),
            full((H1, EM)), full((H1, 1)), full((H1, 1)), full((H1, 1)),
            full((H2, H1)), full((H2, 1)), full((H2, 1)), full((H2, 1)),
            full((H2, 1)), full((1, 1)),
        ],
        out_specs=pl.BlockSpec((1, BT), lambda p, t: (0, jnp.where(p == 2, t, 0))),
        out_shape=jax.ShapeDtypeStruct((1, B), jnp.float32),
        scratch_shapes=[
            pltpu.VMEM((H1, B), jnp.float32),
            pltpu.VMEM((H2, B), jnp.float32),
            pltpu.VMEM((H1, 8), jnp.float32),
            pltpu.VMEM((H2, 8), jnp.float32),
        ],
        compiler_params=pltpu.CompilerParams(
            dimension_semantics=("arbitrary", "arbitrary")),
        interpret=interpret,
    )(embT, W1T, b1, g1, be1, W2T, b2, g2, be2, w3, b3)


def kernel(x, tables, W1, b1, gamma1, beta1, W2, b2, gamma2, beta2, W3, b3):
    xtf = lax.bitcast_convert_type(x.T.astype(jnp.int32), jnp.float32)
    tabT = tables.transpose(0, 2, 1).reshape(EM, V)
    embT = _sc_gather_t(xtf, tabT)
    outT = _mlp_t(embT, W1.T,
                  b1.reshape(H1, 1), gamma1.reshape(H1, 1), beta1.reshape(H1, 1),
                  W2.T, b2.reshape(H2, 1), gamma2.reshape(H2, 1), beta2.reshape(H2, 1),
                  W3, b3.reshape(1, 1))
    return outT[0]


# speedup vs baseline: 1.2378x; 1.2378x over previous
import functools

import jax
import jax.numpy as jnp
from jax import lax
from jax.experimental import pallas as pl
from jax.experimental.pallas import tpu as pltpu
from jax.experimental.pallas import tpu_sc as plsc

V = 100000
NF = 26
D = 32
B = 16384
EM = NF * D
H1 = 128
H2 = 64
NW = 32
CPW = EM // NW
OC = 4096
EPS = 1e-5
BT = 1024
T = B // BT


def _sc_gather_t(xtf, tabT):
    mesh = plsc.VectorSubcoreMesh(core_axis_name="c", subcore_axis_name="s")

    @functools.partial(
        pl.kernel,
        mesh=mesh,
        out_type=jax.ShapeDtypeStruct((EM, B), jnp.float32),
        scratch_types=[
            pltpu.VMEM((B,), jnp.float32),
            pltpu.VMEM((V,), jnp.float32),
            pltpu.VMEM((OC,), jnp.float32),
            pltpu.VMEM((OC,), jnp.float32),
            pltpu.SemaphoreType.DMA,
        ],
        compiler_params=pltpu.CompilerParams(use_tc_tiling_on_sc=True,
                                             needs_layout_passes=False),
    )
    def gk(x_hbm, tab_hbm, out_hbm, idx_v, trow_v, ova, ovb, sem):
        wid = lax.axis_index("s") * 2 + lax.axis_index("c")

        def chan(i, carry):
            c = wid * CPW + i
            f = c // D

            @pl.when(jnp.logical_or(i == 0, f != (c - 1) // D))
            def _():
                pltpu.sync_copy(x_hbm.at[f], idx_v)

            pltpu.sync_copy(tab_hbm.at[c], trow_v)

            def gather_chunk(q):
                buf = ova if q % 2 == 0 else ovb

                def vec(k, inner):
                    for u in range(4):
                        s = k * 64 + u * 16
                        iv = plsc.bitcast(idx_v[pl.ds(q * OC + s, 16)], jnp.int32)
                        buf[pl.ds(s, 16)] = plsc.load_gather(trow_v, [iv])
                    return inner

                lax.fori_loop(0, OC // 64, vec, 0)
                return pltpu.async_copy(
                    buf, out_hbm.at[c, pl.ds(q * OC, OC)], sem)

            hs = [None, None]
            for q in range(B // OC):
                if hs[q % 2] is not None:
                    hs[q % 2].wait()
                hs[q % 2] = gather_chunk(q)
            for h in hs:
                h.wait()
            return carry

        lax.fori_loop(0, CPW, chan, 0)

    return gk(xtf, tabT)


def _mlp_body(emb_ref, w1_ref, b1_ref, g1_ref, be1_ref, w2_ref, b2_ref,
              g2_ref, be2_ref, w3_ref, b3_ref, out_ref,
              h1_ref, h2_ref, acc1_ref, acc2_ref):
    p = pl.program_id(0)
    t = pl.program_id(1)

    @pl.when(jnp.logical_and(p == 0, t == 0))
    def _():
        acc1_ref[...] = jnp.zeros_like(acc1_ref)
        acc2_ref[...] = jnp.zeros_like(acc2_ref)

    @pl.when(p == 0)
    def _():
        h = jnp.dot(w1_ref[...], emb_ref[...], preferred_element_type=jnp.float32)
        h = jnp.maximum(h + b1_ref[...], 0.0)
        h1_ref[:, pl.ds(t * BT, BT)] = h
        acc1_ref[:, 0:1] += jnp.sum(h, axis=1, keepdims=True)
        acc1_ref[:, 1:2] += jnp.sum(h * h, axis=1, keepdims=True)

    @pl.when(jnp.logical_and(p == 1, t == 0))
    def _():
        mean = acc1_ref[:, 0:1] * (1.0 / B)
        var = acc1_ref[:, 1:2] * (1.0 / B) - mean * mean
        sc = g1_ref[...] * lax.rsqrt(var + EPS)
        acc1_ref[:, 2:3] = sc
        acc1_ref[:, 3:4] = be1_ref[...] - mean * sc

    @pl.when(p == 1)
    def _():
        h1n = h1_ref[:, pl.ds(t * BT, BT)] * acc1_ref[:, 2:3] + acc1_ref[:, 3:4]
        h = jnp.dot(w2_ref[...], h1n, preferred_element_type=jnp.float32)
        h = jnp.maximum(h + b2_ref[...], 0.0)
        h2_ref[:, pl.ds(t * BT, BT)] = h
        acc2_ref[:, 0:1] += jnp.sum(h, axis=1, keepdims=True)
        acc2_ref[:, 1:2] += jnp.sum(h * h, axis=1, keepdims=True)

    @pl.when(jnp.logical_and(p == 2, t == 0))
    def _():
        mean = acc2_ref[:, 0:1] * (1.0 / B)
        var = acc2_ref[:, 1:2] * (1.0 / B) - mean * mean
        sc = g2_ref[...] * lax.rsqrt(var + EPS)
        acc2_ref[:, 2:3] = sc
        acc2_ref[:, 3:4] = be2_ref[...] - mean * sc

    @pl.when(p == 2)
    def _():
        h2n = h2_ref[:, pl.ds(t * BT, BT)] * acc2_ref[:, 2:3] + acc2_ref[:, 3:4]
        o = jnp.sum(h2n * w3_ref[...], axis=0, keepdims=True) + b3_ref[...]
        out_ref[...] = o


def _mlp_t(embT, W1T, b1, g1, be1, W2T, b2, g2, be2, w3, b3, interpret=False):
    full = lambda shape: pl.BlockSpec(shape, lambda p, t: (0, 0))
    return pl.pallas_call(
        _mlp_body,
        grid=(3, T),
        in_specs=[
            pl.BlockSpec((EM, BT), lambda p, t: (0, jnp.where(p == 0, t, 0))),
            full((H1, EM)), full((H1, 1)), full((H1, 1)), full((H1, 1)),
            full((H2, H1)), full((H2, 1)), full((H2, 1)), full((H2, 1)),
            full((H2, 1)), full((1, 1)),
        ],
        out_specs=pl.BlockSpec((1, BT), lambda p, t: (0, jnp.where(p == 2, t, 0))),
        out_shape=jax.ShapeDtypeStruct((1, B), jnp.float32),
        scratch_shapes=[
            pltpu.VMEM((H1, B), jnp.float32),
            pltpu.VMEM((H2, B), jnp.float32),
            pltpu.VMEM((H1, 8), jnp.float32),
            pltpu.VMEM((H2, 8), jnp.float32),
        ],
        compiler_params=pltpu.CompilerParams(
            dimension_semantics=("arbitrary", "arbitrary")),
        interpret=interpret,
    )(embT, W1T, b1, g1, be1, W2T, b2, g2, be2, w3, b3)


def kernel(x, tables, W1, b1, gamma1, beta1, W2, b2, gamma2, beta2, W3, b3):
    xtf = lax.bitcast_convert_type(x.T.astype(jnp.int32), jnp.float32)
    tabT = tables.transpose(0, 2, 1).reshape(EM, V)
    embT = _sc_gather_t(xtf, tabT)
    outT = _mlp_t(embT, W1.T,
                  b1.reshape(H1, 1), gamma1.reshape(H1, 1), beta1.reshape(H1, 1),
                  W2.T, b2.reshape(H2, 1), gamma2.reshape(H2, 1), beta2.reshape(H2, 1),
                  W3, b3.reshape(1, 1))
    return outT[0]


# speedup vs baseline: 1.2387x; 1.0007x over previous
import functools

import jax
import jax.numpy as jnp
from jax import lax
from jax.experimental import pallas as pl
from jax.experimental.pallas import tpu as pltpu
from jax.experimental.pallas import tpu_sc as plsc

V = 100000
NF = 26
D = 32
B = 16384
EM = NF * D
H1 = 128
H2 = 64
NW = 32
CPW = EM // NW
OC = 4096
EPS = 1e-5
BT = 1024
T = B // BT


def _sc_gather_t(xtf, tabT):
    mesh = plsc.VectorSubcoreMesh(core_axis_name="c", subcore_axis_name="s")

    @functools.partial(
        pl.kernel,
        mesh=mesh,
        out_type=jax.ShapeDtypeStruct((EM, B), jnp.float32),
        scratch_types=[
            pltpu.VMEM((B,), jnp.float32),
            pltpu.VMEM((V,), jnp.float32),
            pltpu.VMEM((OC,), jnp.float32),
            pltpu.VMEM((OC,), jnp.float32),
            pltpu.VMEM((160,), jnp.float32),
            pltpu.SemaphoreType.DMA,
            pltpu.SemaphoreType.DMA,
        ],
        compiler_params=pltpu.CompilerParams(use_tc_tiling_on_sc=True,
                                             needs_layout_passes=False),
    )
    def gk(x_hbm, tab_hbm, out_hbm, idx_v, trow_v, ova, ovb, tail_v, sem, rsem):
        wid = lax.axis_index("s") * 2 + lax.axis_index("c")
        QS = [(0, 24960), (24960, 24960), (49920, 24960), (74880, 24960)]
        TO = 99840

        def chan(i, carry):
            c = wid * CPW + i
            f = c // D

            @pl.when(jnp.logical_or(i == 0, f != (c - 1) // D))
            def _():
                pltpu.sync_copy(x_hbm.at[f], idx_v)

            rhs = [
                pltpu.async_copy(tab_hbm.at[c, pl.ds(o, s)],
                                 trow_v.at[pl.ds(o, s)], rsem)
                for (o, s) in QS
            ]
            rhs.append(pltpu.async_copy(
                tab_hbm.at[c, pl.ds(TO, V - TO)], tail_v, rsem))
            for rh in rhs:
                rh.wait()
            for u in range((V - TO) // 16):
                trow_v[pl.ds(TO + u * 16, 16)] = tail_v[pl.ds(u * 16, 16)]

            def gather_chunk(q):
                buf = ova if q % 2 == 0 else ovb

                def vec(k, inner):
                    for u in range(4):
                        s = k * 64 + u * 16
                        iv = plsc.bitcast(idx_v[pl.ds(q * OC + s, 16)], jnp.int32)
                        buf[pl.ds(s, 16)] = plsc.load_gather(trow_v, [iv])
                    return inner

                lax.fori_loop(0, OC // 64, vec, 0)
                return pltpu.async_copy(
                    buf, out_hbm.at[c, pl.ds(q * OC, OC)], sem)

            hs = [None, None]
            for q in range(B // OC):
                if hs[q % 2] is not None:
                    hs[q % 2].wait()
                hs[q % 2] = gather_chunk(q)
            for h in hs:
                h.wait()
            return carry

        lax.fori_loop(0, CPW, chan, 0)

    return gk(xtf, tabT)


def _mlp_body(emb_ref, w1_ref, b1_ref, g1_ref, be1_ref, w2_ref, b2_ref,
              g2_ref, be2_ref, w3_ref, b3_ref, out_ref,
              h1_ref, h2_ref, acc1_ref, acc2_ref):
    p = pl.program_id(0)
    t = pl.program_id(1)

    @pl.when(jnp.logical_and(p == 0, t == 0))
    def _():
        acc1_ref[...] = jnp.zeros_like(acc1_ref)
        acc2_ref[...] = jnp.zeros_like(acc2_ref)

    @pl.when(p == 0)
    def _():
        h = jnp.dot(w1_ref[...], emb_ref[...], preferred_element_type=jnp.float32)
        h = jnp.maximum(h + b1_ref[...], 0.0)
        h1_ref[:, pl.ds(t * BT, BT)] = h
        acc1_ref[:, 0:1] += jnp.sum(h, axis=1, keepdims=True)
        acc1_ref[:, 1:2] += jnp.sum(h * h, axis=1, keepdims=True)

    @pl.when(jnp.logical_and(p == 1, t == 0))
    def _():
        mean = acc1_ref[:, 0:1] * (1.0 / B)
        var = acc1_ref[:, 1:2] * (1.0 / B) - mean * mean
        sc = g1_ref[...] * lax.rsqrt(var + EPS)
        acc1_ref[:, 2:3] = sc
        acc1_ref[:, 3:4] = be1_ref[...] - mean * sc

    @pl.when(p == 1)
    def _():
        h1n = h1_ref[:, pl.ds(t * BT, BT)] * acc1_ref[:, 2:3] + acc1_ref[:, 3:4]
        h = jnp.dot(w2_ref[...], h1n, preferred_element_type=jnp.float32)
        h = jnp.maximum(h + b2_ref[...], 0.0)
        h2_ref[:, pl.ds(t * BT, BT)] = h
        acc2_ref[:, 0:1] += jnp.sum(h, axis=1, keepdims=True)
        acc2_ref[:, 1:2] += jnp.sum(h * h, axis=1, keepdims=True)

    @pl.when(jnp.logical_and(p == 2, t == 0))
    def _():
        mean = acc2_ref[:, 0:1] * (1.0 / B)
        var = acc2_ref[:, 1:2] * (1.0 / B) - mean * mean
        sc = g2_ref[...] * lax.rsqrt(var + EPS)
        acc2_ref[:, 2:3] = sc
        acc2_ref[:, 3:4] = be2_ref[...] - mean * sc

    @pl.when(p == 2)
    def _():
        h2n = h2_ref[:, pl.ds(t * BT, BT)] * acc2_ref[:, 2:3] + acc2_ref[:, 3:4]
        o = jnp.sum(h2n * w3_ref[...], axis=0, keepdims=True) + b3_ref[...]
        out_ref[...] = o


def _mlp_t(embT, W1T, b1, g1, be1, W2T, b2, g2, be2, w3, b3, interpret=False):
    full = lambda shape: pl.BlockSpec(shape, lambda p, t: (0, 0))
    return pl.pallas_call(
        _mlp_body,
        grid=(3, T),
        in_specs=[
            pl.BlockSpec((EM, BT), lambda p, t: (0, jnp.where(p == 0, t, 0))),
            full((H1, EM)), full((H1, 1)), full((H1, 1)), full((H1, 1)),
            full((H2, H1)), full((H2, 1)), full((H2, 1)), full((H2, 1)),
            full((H2, 1)), full((1, 1)),
        ],
        out_specs=pl.BlockSpec((1, BT), lambda p, t: (0, jnp.where(p == 2, t, 0))),
        out_shape=jax.ShapeDtypeStruct((1, B), jnp.float32),
        scratch_shapes=[
            pltpu.VMEM((H1, B), jnp.float32),
            pltpu.VMEM((H2, B), jnp.float32),
            pltpu.VMEM((H1, 8), jnp.float32),
            pltpu.VMEM((H2, 8), jnp.float32),
        ],
        compiler_params=pltpu.CompilerParams(
            dimension_semantics=("arbitrary", "arbitrary")),
        interpret=interpret,
    )(embT, W1T, b1, g1, be1, W2T, b2, g2, be2, w3, b3)


def kernel(x, tables, W1, b1, gamma1, beta1, W2, b2, gamma2, beta2, W3, b3):
    xtf = lax.bitcast_convert_type(x.T.astype(jnp.int32), jnp.float32)
    tabT = tables.transpose(0, 2, 1).reshape(EM, V)
    embT = _sc_gather_t(xtf, tabT)
    outT = _mlp_t(embT, W1.T,
                  b1.reshape(H1, 1), gamma1.reshape(H1, 1), beta1.reshape(H1, 1),
                  W2.T, b2.reshape(H2, 1), gamma2.reshape(H2, 1), beta2.reshape(H2, 1),
                  W3, b3.reshape(1, 1))
    return outT[0]


# speedup vs baseline: 2.1738x; 1.7549x over previous
import functools

import jax
import jax.numpy as jnp
from jax import lax
from jax.experimental import pallas as pl
from jax.experimental.pallas import tpu as pltpu
from jax.experimental.pallas import tpu_sc as plsc

V = 100000
NF = 26
D = 32
B = 16384
EM = NF * D
H1 = 128
H2 = 64
NW = 32
CPW = EM // NW
OC = 4096
EPS = 1e-5
BT = 1024
T = B // BT


def _sc_gather_t(xtf, tabT):
    mesh = plsc.VectorSubcoreMesh(core_axis_name="c", subcore_axis_name="s")

    @functools.partial(
        pl.kernel,
        mesh=mesh,
        out_type=jax.ShapeDtypeStruct((EM, B), jnp.float32),
        scratch_types=[
            pltpu.VMEM((B,), jnp.float32),
            pltpu.VMEM((V,), jnp.float32),
            pltpu.VMEM((OC,), jnp.float32),
            pltpu.VMEM((OC,), jnp.float32),
            pltpu.VMEM((160,), jnp.float32),
            pltpu.SemaphoreType.DMA,
            pltpu.SemaphoreType.DMA,
        ],
        compiler_params=pltpu.CompilerParams(use_tc_tiling_on_sc=True,
                                             needs_layout_passes=False),
    )
    def gk(x_hbm, tab_hbm, out_hbm, idx_v, trow_v, ova, ovb, tail_v, sem, rsem):
        wid = lax.axis_index("s") * 2 + lax.axis_index("c")
        QS = [(0, 24960), (24960, 24960), (49920, 24960), (74880, 24960)]
        TO = 99840

        def chan(i, carry):
            c = wid * CPW + i
            f = c // D

            @pl.when(jnp.logical_or(i == 0, f != (c - 1) // D))
            def _():
                pltpu.sync_copy(x_hbm.at[f], idx_v)

            rhs = [
                pltpu.async_copy(tab_hbm.at[c, pl.ds(o, s)],
                                 trow_v.at[pl.ds(o, s)], rsem)
                for (o, s) in QS
            ]
            rhs.append(pltpu.async_copy(
                tab_hbm.at[c, pl.ds(TO, V - TO)], tail_v, rsem))
            for rh in rhs:
                rh.wait()
            for u in range((V - TO) // 16):
                trow_v[pl.ds(TO + u * 16, 16)] = tail_v[pl.ds(u * 16, 16)]

            def gather_chunk(q):
                buf = ova if q % 2 == 0 else ovb

                @plsc.parallel_loop(0, OC, step=16, unroll=8)
                def _(s):
                    iv = plsc.bitcast(idx_v[pl.ds(q * OC + s, 16)], jnp.int32)
                    buf[pl.ds(s, 16)] = plsc.load_gather(trow_v, [iv])

                return pltpu.async_copy(
                    buf, out_hbm.at[c, pl.ds(q * OC, OC)], sem)

            hs = [None, None]
            for q in range(B // OC):
                if hs[q % 2] is not None:
                    hs[q % 2].wait()
                hs[q % 2] = gather_chunk(q)
            for h in hs:
                h.wait()
            return carry

        lax.fori_loop(0, CPW, chan, 0)

    return gk(xtf, tabT)


def _mlp_body(emb_ref, w1_ref, b1_ref, g1_ref, be1_ref, w2_ref, b2_ref,
              g2_ref, be2_ref, w3_ref, b3_ref, out_ref,
              h1_ref, h2_ref, acc1_ref, acc2_ref):
    p = pl.program_id(0)
    t = pl.program_id(1)

    @pl.when(jnp.logical_and(p == 0, t == 0))
    def _():
        acc1_ref[...] = jnp.zeros_like(acc1_ref)
        acc2_ref[...] = jnp.zeros_like(acc2_ref)

    @pl.when(p == 0)
    def _():
        h = jnp.dot(w1_ref[...], emb_ref[...], preferred_element_type=jnp.float32)
        h = jnp.maximum(h + b1_ref[...], 0.0)
        h1_ref[:, pl.ds(t * BT, BT)] = h
        acc1_ref[:, 0:1] += jnp.sum(h, axis=1, keepdims=True)
        acc1_ref[:, 1:2] += jnp.sum(h * h, axis=1, keepdims=True)

    @pl.when(jnp.logical_and(p == 1, t == 0))
    def _():
        mean = acc1_ref[:, 0:1] * (1.0 / B)
        var = acc1_ref[:, 1:2] * (1.0 / B) - mean * mean
        sc = g1_ref[...] * lax.rsqrt(var + EPS)
        acc1_ref[:, 2:3] = sc
        acc1_ref[:, 3:4] = be1_ref[...] - mean * sc

    @pl.when(p == 1)
    def _():
        h1n = h1_ref[:, pl.ds(t * BT, BT)] * acc1_ref[:, 2:3] + acc1_ref[:, 3:4]
        h = jnp.dot(w2_ref[...], h1n, preferred_element_type=jnp.float32)
        h = jnp.maximum(h + b2_ref[...], 0.0)
        h2_ref[:, pl.ds(t * BT, BT)] = h
        acc2_ref[:, 0:1] += jnp.sum(h, axis=1, keepdims=True)
        acc2_ref[:, 1:2] += jnp.sum(h * h, axis=1, keepdims=True)

    @pl.when(jnp.logical_and(p == 2, t == 0))
    def _():
        mean = acc2_ref[:, 0:1] * (1.0 / B)
        var = acc2_ref[:, 1:2] * (1.0 / B) - mean * mean
        sc = g2_ref[...] * lax.rsqrt(var + EPS)
        acc2_ref[:, 2:3] = sc
        acc2_ref[:, 3:4] = be2_ref[...] - mean * sc

    @pl.when(p == 2)
    def _():
        h2n = h2_ref[:, pl.ds(t * BT, BT)] * acc2_ref[:, 2:3] + acc2_ref[:, 3:4]
        o = jnp.sum(h2n * w3_ref[...], axis=0, keepdims=True) + b3_ref[...]
        out_ref[...] = o


def _mlp_t(embT, W1T, b1, g1, be1, W2T, b2, g2, be2, w3, b3, interpret=False):
    full = lambda shape: pl.BlockSpec(shape, lambda p, t: (0, 0))
    return pl.pallas_call(
        _mlp_body,
        grid=(3, T),
        in_specs=[
            pl.BlockSpec((EM, BT), lambda p, t: (0, jnp.where(p == 0, t, 0))),
            full((H1, EM)), full((H1, 1)), full((H1, 1)), full((H1, 1)),
            full((H2, H1)), full((H2, 1)), full((H2, 1)), full((H2, 1)),
            full((H2, 1)), full((1, 1)),
        ],
        out_specs=pl.BlockSpec((1, BT), lambda p, t: (0, jnp.where(p == 2, t, 0))),
        out_shape=jax.ShapeDtypeStruct((1, B), jnp.float32),
        scratch_shapes=[
            pltpu.VMEM((H1, B), jnp.float32),
            pltpu.VMEM((H2, B), jnp.float32),
            pltpu.VMEM((H1, 8), jnp.float32),
            pltpu.VMEM((H2, 8), jnp.float32),
        ],
        compiler_params=pltpu.CompilerParams(
            dimension_semantics=("arbitrary", "arbitrary")),
        interpret=interpret,
    )(embT, W1T, b1, g1, be1, W2T, b2, g2, be2, w3, b3)


def kernel(x, tables, W1, b1, gamma1, beta1, W2, b2, gamma2, beta2, W3, b3):
    xtf = lax.bitcast_convert_type(x.T.astype(jnp.int32), jnp.float32)
    tabT = tables.transpose(0, 2, 1).reshape(EM, V)
    embT = _sc_gather_t(xtf, tabT)
    outT = _mlp_t(embT, W1.T,
                  b1.reshape(H1, 1), gamma1.reshape(H1, 1), beta1.reshape(H1, 1),
                  W2.T, b2.reshape(H2, 1), gamma2.reshape(H2, 1), beta2.reshape(H2, 1),
                  W3, b3.reshape(1, 1))
    return outT[0]


# speedup vs baseline: 2.2740x; 1.0461x over previous
import functools

import jax
import jax.numpy as jnp
from jax import lax
from jax.experimental import pallas as pl
from jax.experimental.pallas import tpu as pltpu
from jax.experimental.pallas import tpu_sc as plsc

V = 100000
NF = 26
D = 32
B = 16384
EM = NF * D
H1 = 128
H2 = 64
NW = 32
CPW = EM // NW
OC = 4096
EPS = 1e-5
BT = 2048
T = B // BT


def _sc_gather_t(xtf, tabT):
    mesh = plsc.VectorSubcoreMesh(core_axis_name="c", subcore_axis_name="s")

    @functools.partial(
        pl.kernel,
        mesh=mesh,
        out_type=jax.ShapeDtypeStruct((EM, B), jnp.float32),
        scratch_types=[
            pltpu.VMEM((B,), jnp.float32),
            pltpu.VMEM((V,), jnp.float32),
            pltpu.VMEM((OC,), jnp.float32),
            pltpu.VMEM((OC,), jnp.float32),
            pltpu.VMEM((160,), jnp.float32),
            pltpu.SemaphoreType.DMA,
            pltpu.SemaphoreType.DMA,
        ],
        compiler_params=pltpu.CompilerParams(use_tc_tiling_on_sc=True,
                                             needs_layout_passes=False),
    )
    def gk(x_hbm, tab_hbm, out_hbm, idx_v, trow_v, ova, ovb, tail_v, sem, rsem):
        wid = lax.axis_index("s") * 2 + lax.axis_index("c")
        QS = [(0, 24960), (24960, 24960), (49920, 24960), (74880, 24960)]
        TO = 99840

        def chan(i, carry):
            c = wid * CPW + i
            f = c // D

            @pl.when(jnp.logical_or(i == 0, f != (c - 1) // D))
            def _():
                pltpu.sync_copy(x_hbm.at[f], idx_v)

            rhs = [
                pltpu.async_copy(tab_hbm.at[c, pl.ds(o, s)],
                                 trow_v.at[pl.ds(o, s)], rsem)
                for (o, s) in QS
            ]
            rhs.append(pltpu.async_copy(
                tab_hbm.at[c, pl.ds(TO, V - TO)], tail_v, rsem))
            for rh in rhs:
                rh.wait()
            for u in range((V - TO) // 16):
                trow_v[pl.ds(TO + u * 16, 16)] = tail_v[pl.ds(u * 16, 16)]

            def gather_chunk(q):
                buf = ova if q % 2 == 0 else ovb

                @plsc.parallel_loop(0, OC, step=16, unroll=8)
                def _(s):
                    iv = plsc.bitcast(idx_v[pl.ds(q * OC + s, 16)], jnp.int32)
                    buf[pl.ds(s, 16)] = plsc.load_gather(trow_v, [iv])

                return pltpu.async_copy(
                    buf, out_hbm.at[c, pl.ds(q * OC, OC)], sem)

            hs = [None, None]
            for q in range(B // OC):
                if hs[q % 2] is not None:
                    hs[q % 2].wait()
                hs[q % 2] = gather_chunk(q)
            for h in hs:
                h.wait()
            return carry

        lax.fori_loop(0, CPW, chan, 0)

    return gk(xtf, tabT)


def _mlp_body(emb_ref, w1_ref, b1_ref, g1_ref, be1_ref, w2_ref, b2_ref,
              g2_ref, be2_ref, w3_ref, b3_ref, out_ref,
              h1_ref, h2_ref, acc1_ref, acc2_ref):
    p = pl.program_id(0)
    t = pl.program_id(1)

    @pl.when(jnp.logical_and(p == 0, t == 0))
    def _():
        acc1_ref[...] = jnp.zeros_like(acc1_ref)
        acc2_ref[...] = jnp.zeros_like(acc2_ref)

    @pl.when(p == 0)
    def _():
        h = jnp.dot(w1_ref[...], emb_ref[...], preferred_element_type=jnp.float32)
        h = jnp.maximum(h + b1_ref[...], 0.0)
        h1_ref[:, pl.ds(t * BT, BT)] = h
        acc1_ref[:, 0:1] += jnp.sum(h, axis=1, keepdims=True)
        acc1_ref[:, 1:2] += jnp.sum(h * h, axis=1, keepdims=True)

    @pl.when(jnp.logical_and(p == 1, t == 0))
    def _():
        mean = acc1_ref[:, 0:1] * (1.0 / B)
        var = acc1_ref[:, 1:2] * (1.0 / B) - mean * mean
        sc = g1_ref[...] * lax.rsqrt(var + EPS)
        acc1_ref[:, 2:3] = sc
        acc1_ref[:, 3:4] = be1_ref[...] - mean * sc

    @pl.when(p == 1)
    def _():
        h1n = h1_ref[:, pl.ds(t * BT, BT)] * acc1_ref[:, 2:3] + acc1_ref[:, 3:4]
        h = jnp.dot(w2_ref[...], h1n, preferred_element_type=jnp.float32)
        h = jnp.maximum(h + b2_ref[...], 0.0)
        h2_ref[:, pl.ds(t * BT, BT)] = h
        acc2_ref[:, 0:1] += jnp.sum(h, axis=1, keepdims=True)
        acc2_ref[:, 1:2] += jnp.sum(h * h, axis=1, keepdims=True)

    @pl.when(jnp.logical_and(p == 2, t == 0))
    def _():
        mean = acc2_ref[:, 0:1] * (1.0 / B)
        var = acc2_ref[:, 1:2] * (1.0 / B) - mean * mean
        sc = g2_ref[...] * lax.rsqrt(var + EPS)
        acc2_ref[:, 2:3] = sc
        acc2_ref[:, 3:4] = be2_ref[...] - mean * sc

    @pl.when(p == 2)
    def _():
        h2n = h2_ref[:, pl.ds(t * BT, BT)] * acc2_ref[:, 2:3] + acc2_ref[:, 3:4]
        o = jnp.sum(h2n * w3_ref[...], axis=0, keepdims=True) + b3_ref[...]
        out_ref[...] = o


def _mlp_t(embT, W1T, b1, g1, be1, W2T, b2, g2, be2, w3, b3, interpret=False):
    full = lambda shape: pl.BlockSpec(shape, lambda p, t: (0, 0))
    return pl.pallas_call(
        _mlp_body,
        grid=(3, T),
        in_specs=[
            pl.BlockSpec((EM, BT), lambda p, t: (0, jnp.where(p == 0, t, 0))),
            full((H1, EM)), full((H1, 1)), full((H1, 1)), full((H1, 1)),
            full((H2, H1)), full((H2, 1)), full((H2, 1)), full((H2, 1)),
            full((H2, 1)), full((1, 1)),
        ],
        out_specs=pl.BlockSpec((1, BT), lambda p, t: (0, jnp.where(p == 2, t, 0))),
        out_shape=jax.ShapeDtypeStruct((1, B), jnp.float32),
        scratch_shapes=[
            pltpu.VMEM((H1, B), jnp.float32),
            pltpu.VMEM((H2, B), jnp.float32),
            pltpu.VMEM((H1, 8), jnp.float32),
            pltpu.VMEM((H2, 8), jnp.float32),
        ],
        compiler_params=pltpu.CompilerParams(
            dimension_semantics=("arbitrary", "arbitrary")),
        interpret=interpret,
    )(embT, W1T, b1, g1, be1, W2T, b2, g2, be2, w3, b3)


def kernel(x, tables, W1, b1, gamma1, beta1, W2, b2, gamma2, beta2, W3, b3):
    xtf = lax.bitcast_convert_type(x.T.astype(jnp.int32), jnp.float32)
    tabT = tables.transpose(0, 2, 1).reshape(EM, V)
    embT = _sc_gather_t(xtf, tabT)
    outT = _mlp_t(embT, W1.T,
                  b1.reshape(H1, 1), gamma1.reshape(H1, 1), beta1.reshape(H1, 1),
                  W2.T, b2.reshape(H2, 1), gamma2.reshape(H2, 1), beta2.reshape(H2, 1),
                  W3, b3.reshape(1, 1))
    return outT[0]


# speedup vs baseline: 2.2787x; 1.0021x over previous
import functools

import jax
import jax.numpy as jnp
from jax import lax
from jax.experimental import pallas as pl
from jax.experimental.pallas import tpu as pltpu
from jax.experimental.pallas import tpu_sc as plsc

V = 100000
NF = 26
D = 32
B = 16384
EM = NF * D
H1 = 128
H2 = 64
NW = 32
CPW = EM // NW
OC = 4096
EPS = 1e-5
BT = 2048
T = B // BT


def _sc_gather_t(xtf, tabT):
    mesh = plsc.VectorSubcoreMesh(core_axis_name="c", subcore_axis_name="s")

    @functools.partial(
        pl.kernel,
        mesh=mesh,
        out_type=jax.ShapeDtypeStruct((EM, B), jnp.float32),
        scratch_types=[
            pltpu.VMEM((B,), jnp.float32),
            pltpu.VMEM((V,), jnp.float32),
            pltpu.VMEM((OC,), jnp.float32),
            pltpu.VMEM((OC,), jnp.float32),
            pltpu.VMEM((160,), jnp.float32),
            pltpu.SemaphoreType.DMA,
            pltpu.SemaphoreType.DMA,
        ],
        compiler_params=pltpu.CompilerParams(use_tc_tiling_on_sc=True,
                                             needs_layout_passes=False),
    )
    def gk(x_hbm, tab_hbm, out_hbm, idx_v, trow_v, ova, ovb, tail_v, sem, rsem):
        wid = lax.axis_index("s") * 2 + lax.axis_index("c")
        QS = [(0, 24960), (24960, 24960), (49920, 24960), (74880, 24960)]
        TO = 99840

        def chan(i, carry):
            c = wid * CPW + i
            f = c // D

            @pl.when(jnp.logical_or(i == 0, f != (c - 1) // D))
            def _():
                pltpu.sync_copy(x_hbm.at[f], idx_v)

            rhs = [
                pltpu.async_copy(tab_hbm.at[c, pl.ds(o, s)],
                                 trow_v.at[pl.ds(o, s)], rsem)
                for (o, s) in QS
            ]
            rhs.append(pltpu.async_copy(
                tab_hbm.at[c, pl.ds(TO, V - TO)], tail_v, rsem))
            for rh in rhs:
                rh.wait()
            for u in range((V - TO) // 16):
                trow_v[pl.ds(TO + u * 16, 16)] = tail_v[pl.ds(u * 16, 16)]

            def gather_chunk(q):
                buf = ova if q % 2 == 0 else ovb

                @plsc.parallel_loop(0, OC, step=16, unroll=8)
                def _(s):
                    iv = plsc.bitcast(idx_v[pl.ds(q * OC + s, 16)], jnp.int32)
                    buf[pl.ds(s, 16)] = plsc.load_gather(trow_v, [iv])

                return pltpu.async_copy(
                    buf, out_hbm.at[c, pl.ds(q * OC, OC)], sem)

            hs = [None, None]
            for q in range(B // OC):
                if hs[q % 2] is not None:
                    hs[q % 2].wait()
                hs[q % 2] = gather_chunk(q)
            for h in hs:
                h.wait()
            return carry

        lax.fori_loop(0, CPW, chan, 0)

    return gk(xtf, tabT)


def _mlp_body(emb_ref, w1_ref, b1_ref, g1_ref, be1_ref, w2_ref, b2_ref,
              g2_ref, be2_ref, w3_ref, b3_ref, out_ref,
              h1_ref, h2_ref, acc1_ref, acc2_ref):
    p = pl.program_id(0)
    t = pl.program_id(1)

    @pl.when(jnp.logical_and(p == 0, t == 0))
    def _():
        acc1_ref[...] = jnp.zeros_like(acc1_ref)
        acc2_ref[...] = jnp.zeros_like(acc2_ref)

    @pl.when(p == 0)
    def _():
        h = jnp.dot(w1_ref[...].astype(jnp.bfloat16),
                    emb_ref[...].astype(jnp.bfloat16),
                    preferred_element_type=jnp.float32)
        h = jnp.maximum(h + b1_ref[...], 0.0)
        h1_ref[:, pl.ds(t * BT, BT)] = h
        acc1_ref[:, 0:1] += jnp.sum(h, axis=1, keepdims=True)
        acc1_ref[:, 1:2] += jnp.sum(h * h, axis=1, keepdims=True)

    @pl.when(jnp.logical_and(p == 1, t == 0))
    def _():
        mean = acc1_ref[:, 0:1] * (1.0 / B)
        var = acc1_ref[:, 1:2] * (1.0 / B) - mean * mean
        sc = g1_ref[...] * lax.rsqrt(var + EPS)
        acc1_ref[:, 2:3] = sc
        acc1_ref[:, 3:4] = be1_ref[...] - mean * sc

    @pl.when(p == 1)
    def _():
        h1n = h1_ref[:, pl.ds(t * BT, BT)] * acc1_ref[:, 2:3] + acc1_ref[:, 3:4]
        h = jnp.dot(w2_ref[...].astype(jnp.bfloat16), h1n.astype(jnp.bfloat16),
                    preferred_element_type=jnp.float32)
        h = jnp.maximum(h + b2_ref[...], 0.0)
        h2_ref[:, pl.ds(t * BT, BT)] = h
        acc2_ref[:, 0:1] += jnp.sum(h, axis=1, keepdims=True)
        acc2_ref[:, 1:2] += jnp.sum(h * h, axis=1, keepdims=True)

    @pl.when(jnp.logical_and(p == 2, t == 0))
    def _():
        mean = acc2_ref[:, 0:1] * (1.0 / B)
        var = acc2_ref[:, 1:2] * (1.0 / B) - mean * mean
        sc = g2_ref[...] * lax.rsqrt(var + EPS)
        acc2_ref[:, 2:3] = sc
        acc2_ref[:, 3:4] = be2_ref[...] - mean * sc

    @pl.when(p == 2)
    def _():
        h2n = h2_ref[:, pl.ds(t * BT, BT)] * acc2_ref[:, 2:3] + acc2_ref[:, 3:4]
        o = jnp.sum(h2n * w3_ref[...], axis=0, keepdims=True) + b3_ref[...]
        out_ref[...] = o


def _mlp_t(embT, W1T, b1, g1, be1, W2T, b2, g2, be2, w3, b3, interpret=False):
    full = lambda shape: pl.BlockSpec(shape, lambda p, t: (0, 0))
    return pl.pallas_call(
        _mlp_body,
        grid=(3, T),
        in_specs=[
            pl.BlockSpec((EM, BT), lambda p, t: (0, jnp.where(p == 0, t, 0))),
            full((H1, EM)), full((H1, 1)), full((H1, 1)), full((H1, 1)),
            full((H2, H1)), full((H2, 1)), full((H2, 1)), full((H2, 1)),
            full((H2, 1)), full((1, 1)),
        ],
        out_specs=pl.BlockSpec((1, BT), lambda p, t: (0, jnp.where(p == 2, t, 0))),
        out_shape=jax.ShapeDtypeStruct((1, B), jnp.float32),
        scratch_shapes=[
            pltpu.VMEM((H1, B), jnp.float32),
            pltpu.VMEM((H2, B), jnp.float32),
            pltpu.VMEM((H1, 8), jnp.float32),
            pltpu.VMEM((H2, 8), jnp.float32),
        ],
        compiler_params=pltpu.CompilerParams(
            dimension_semantics=("arbitrary", "arbitrary")),
        interpret=interpret,
    )(embT, W1T, b1, g1, be1, W2T, b2, g2, be2, w3, b3)


def kernel(x, tables, W1, b1, gamma1, beta1, W2, b2, gamma2, beta2, W3, b3):
    xtf = lax.bitcast_convert_type(x.T.astype(jnp.int32), jnp.float32)
    tabT = tables.transpose(0, 2, 1).reshape(EM, V)
    embT = _sc_gather_t(xtf, tabT)
    outT = _mlp_t(embT, W1.T,
                  b1.reshape(H1, 1), gamma1.reshape(H1, 1), beta1.reshape(H1, 1),
                  W2.T, b2.reshape(H2, 1), gamma2.reshape(H2, 1), beta2.reshape(H2, 1),
                  W3, b3.reshape(1, 1))
    return outT[0]


# speedup vs baseline: 2.2793x; 1.0003x over previous
import functools

import jax
import jax.numpy as jnp
from jax import lax
from jax.experimental import pallas as pl
from jax.experimental.pallas import tpu as pltpu
from jax.experimental.pallas import tpu_sc as plsc

V = 100000
NF = 26
D = 32
B = 16384
EM = NF * D
H1 = 128
H2 = 64
NW = 32
CPW = EM // NW
OC = 4096
EPS = 1e-5
BT = 2048
T = B // BT


def _sc_gather_t(xtf, tabT):
    mesh = plsc.VectorSubcoreMesh(core_axis_name="c", subcore_axis_name="s")

    @functools.partial(
        pl.kernel,
        mesh=mesh,
        out_type=jax.ShapeDtypeStruct((EM, B), jnp.float32),
        scratch_types=[
            pltpu.VMEM((B,), jnp.float32),
            pltpu.VMEM((V,), jnp.float32),
            pltpu.VMEM((OC,), jnp.float32),
            pltpu.VMEM((OC,), jnp.float32),
            pltpu.VMEM((160,), jnp.float32),
            pltpu.SemaphoreType.DMA,
            pltpu.SemaphoreType.DMA,
        ],
        compiler_params=pltpu.CompilerParams(use_tc_tiling_on_sc=True,
                                             needs_layout_passes=False),
    )
    def gk(x_hbm, tab_hbm, out_hbm, idx_v, trow_v, ova, ovb, tail_v, sem, rsem):
        wid = lax.axis_index("s") * 2 + lax.axis_index("c")
        QS = [(0, 24960), (24960, 24960), (49920, 24960), (74880, 24960)]
        TO = 99840

        def chan(i, carry):
            c = wid * CPW + i
            f = c // D

            @pl.when(jnp.logical_or(i == 0, f != (c - 1) // D))
            def _():
                pltpu.sync_copy(x_hbm.at[f], idx_v)

            rhs = [
                pltpu.async_copy(tab_hbm.at[c, pl.ds(o, s)],
                                 trow_v.at[pl.ds(o, s)], rsem)
                for (o, s) in QS
            ]
            rhs.append(pltpu.async_copy(
                tab_hbm.at[c, pl.ds(TO, V - TO)], tail_v, rsem))
            for rh in rhs:
                rh.wait()
            for u in range((V - TO) // 16):
                trow_v[pl.ds(TO + u * 16, 16)] = tail_v[pl.ds(u * 16, 16)]

            def gather_chunk(q):
                buf = ova if q % 2 == 0 else ovb

                @plsc.parallel_loop(0, OC, step=16, unroll=16)
                def _(s):
                    iv = plsc.bitcast(idx_v[pl.ds(q * OC + s, 16)], jnp.int32)
                    buf[pl.ds(s, 16)] = plsc.load_gather(trow_v, [iv])

                return pltpu.async_copy(
                    buf, out_hbm.at[c, pl.ds(q * OC, OC)], sem)

            hs = [None, None]
            for q in range(B // OC):
                if hs[q % 2] is not None:
                    hs[q % 2].wait()
                hs[q % 2] = gather_chunk(q)
            for h in hs:
                h.wait()
            return carry

        lax.fori_loop(0, CPW, chan, 0)

    return gk(xtf, tabT)


def _mlp_body(emb_ref, w1_ref, b1_ref, g1_ref, be1_ref, w2_ref, b2_ref,
              g2_ref, be2_ref, w3_ref, b3_ref, out_ref,
              h1_ref, h2_ref, acc1_ref, acc2_ref):
    p = pl.program_id(0)
    t = pl.program_id(1)

    @pl.when(jnp.logical_and(p == 0, t == 0))
    def _():
        acc1_ref[...] = jnp.zeros_like(acc1_ref)
        acc2_ref[...] = jnp.zeros_like(acc2_ref)

    @pl.when(p == 0)
    def _():
        h = jnp.dot(w1_ref[...], emb_ref[...], preferred_element_type=jnp.float32)
        h = jnp.maximum(h + b1_ref[...], 0.0)
        h1_ref[:, pl.ds(t * BT, BT)] = h
        acc1_ref[:, 0:1] += jnp.sum(h, axis=1, keepdims=True)
        acc1_ref[:, 1:2] += jnp.sum(h * h, axis=1, keepdims=True)

    @pl.when(jnp.logical_and(p == 1, t == 0))
    def _():
        mean = acc1_ref[:, 0:1] * (1.0 / B)
        var = acc1_ref[:, 1:2] * (1.0 / B) - mean * mean
        sc = g1_ref[...] * lax.rsqrt(var + EPS)
        acc1_ref[:, 2:3] = sc
        acc1_ref[:, 3:4] = be1_ref[...] - mean * sc

    @pl.when(p == 1)
    def _():
        h1n = h1_ref[:, pl.ds(t * BT, BT)] * acc1_ref[:, 2:3] + acc1_ref[:, 3:4]
        h = jnp.dot(w2_ref[...], h1n, preferred_element_type=jnp.float32)
        h = jnp.maximum(h + b2_ref[...], 0.0)
        h2_ref[:, pl.ds(t * BT, BT)] = h
        acc2_ref[:, 0:1] += jnp.sum(h, axis=1, keepdims=True)
        acc2_ref[:, 1:2] += jnp.sum(h * h, axis=1, keepdims=True)

    @pl.when(jnp.logical_and(p == 2, t == 0))
    def _():
        mean = acc2_ref[:, 0:1] * (1.0 / B)
        var = acc2_ref[:, 1:2] * (1.0 / B) - mean * mean
        sc = g2_ref[...] * lax.rsqrt(var + EPS)
        acc2_ref[:, 2:3] = sc
        acc2_ref[:, 3:4] = be2_ref[...] - mean * sc

    @pl.when(p == 2)
    def _():
        h2n = h2_ref[:, pl.ds(t * BT, BT)] * acc2_ref[:, 2:3] + acc2_ref[:, 3:4]
        o = jnp.sum(h2n * w3_ref[...], axis=0, keepdims=True) + b3_ref[...]
        out_ref[...] = o


def _mlp_t(embT, W1T, b1, g1, be1, W2T, b2, g2, be2, w3, b3, interpret=False):
    full = lambda shape: pl.BlockSpec(shape, lambda p, t: (0, 0))
    return pl.pallas_call(
        _mlp_body,
        grid=(3, T),
        in_specs=[
            pl.BlockSpec((EM, BT), lambda p, t: (0, jnp.where(p == 0, t, 0))),
            full((H1, EM)), full((H1, 1)), full((H1, 1)), full((H1, 1)),
            full((H2, H1)), full((H2, 1)), full((H2, 1)), full((H2, 1)),
            full((H2, 1)), full((1, 1)),
        ],
        out_specs=pl.BlockSpec((1, BT), lambda p, t: (0, jnp.where(p == 2, t, 0))),
        out_shape=jax.ShapeDtypeStruct((1, B), jnp.float32),
        scratch_shapes=[
            pltpu.VMEM((H1, B), jnp.float32),
            pltpu.VMEM((H2, B), jnp.float32),
            pltpu.VMEM((H1, 8), jnp.float32),
            pltpu.VMEM((H2, 8), jnp.float32),
        ],
        compiler_params=pltpu.CompilerParams(
            dimension_semantics=("arbitrary", "arbitrary")),
        interpret=interpret,
    )(embT, W1T, b1, g1, be1, W2T, b2, g2, be2, w3, b3)


def kernel(x, tables, W1, b1, gamma1, beta1, W2, b2, gamma2, beta2, W3, b3):
    xtf = lax.bitcast_convert_type(x.T.astype(jnp.int32), jnp.float32)
    tabT = tables.transpose(0, 2, 1).reshape(EM, V)
    embT = _sc_gather_t(xtf, tabT)
    outT = _mlp_t(embT, W1.T,
                  b1.reshape(H1, 1), gamma1.reshape(H1, 1), beta1.reshape(H1, 1),
                  W2.T, b2.reshape(H2, 1), gamma2.reshape(H2, 1), beta2.reshape(H2, 1),
                  W3, b3.reshape(1, 1))
    return outT[0]


# speedup vs baseline: 2.2871x; 1.0034x over previous
import functools

import jax
import jax.numpy as jnp
from jax import lax
from jax.experimental import pallas as pl
from jax.experimental.pallas import tpu as pltpu
from jax.experimental.pallas import tpu_sc as plsc

V = 100000
NF = 26
D = 32
B = 16384
EM = NF * D
H1 = 128
H2 = 64
NW = 32
CPW = EM // NW
OC = 4096
EPS = 1e-5
BT = 2048
T = B // BT


def _sc_gather_t(xtf, tabT):
    mesh = plsc.VectorSubcoreMesh(core_axis_name="c", subcore_axis_name="s")

    @functools.partial(
        pl.kernel,
        mesh=mesh,
        out_type=jax.ShapeDtypeStruct((EM, B), jnp.float32),
        scratch_types=[
            pltpu.VMEM((B,), jnp.float32),
            pltpu.VMEM((V,), jnp.float32),
            pltpu.VMEM((OC,), jnp.float32),
            pltpu.VMEM((OC,), jnp.float32),
            pltpu.VMEM((160,), jnp.float32),
            pltpu.SemaphoreType.DMA,
            pltpu.SemaphoreType.DMA,
        ],
        compiler_params=pltpu.CompilerParams(use_tc_tiling_on_sc=True,
                                             needs_layout_passes=False),
    )
    def gk(x_hbm, tab_hbm, out_hbm, idx_v, trow_v, ova, ovb, tail_v, sem, rsem):
        wid = lax.axis_index("s") * 2 + lax.axis_index("c")
        QS = [(0, 24960), (24960, 24960), (49920, 24960), (74880, 24960)]
        TO = 99840

        def chan(i, carry):
            c = wid * CPW + i
            f = c // D

            @pl.when(jnp.logical_or(i == 0, f != (c - 1) // D))
            def _():
                pltpu.sync_copy(x_hbm.at[f], idx_v)

            rhs = [
                pltpu.async_copy(tab_hbm.at[c, pl.ds(o, s)],
                                 trow_v.at[pl.ds(o, s)], rsem)
                for (o, s) in QS
            ]
            rhs.append(pltpu.async_copy(
                tab_hbm.at[c, pl.ds(TO, V - TO)], tail_v, rsem))
            for rh in rhs:
                rh.wait()
            for u in range((V - TO) // 16):
                trow_v[pl.ds(TO + u * 16, 16)] = tail_v[pl.ds(u * 16, 16)]

            def gather_chunk(q):
                buf = ova if q % 2 == 0 else ovb

                @plsc.parallel_loop(0, OC, step=16, unroll=16)
                def _(s):
                    iv = plsc.bitcast(idx_v[pl.ds(q * OC + s, 16)], jnp.int32)
                    buf[pl.ds(s, 16)] = plsc.load_gather(trow_v, [iv])

                return pltpu.async_copy(
                    buf, out_hbm.at[c, pl.ds(q * OC, OC)], sem)

            hs = [None, None]
            for q in range(B // OC):
                if hs[q % 2] is not None:
                    hs[q % 2].wait()
                hs[q % 2] = gather_chunk(q)
            for h in hs:
                h.wait()
            return carry

        lax.fori_loop(0, CPW, chan, 0)

    return gk(xtf, tabT)


def _mlp_body(emb_ref, w1_ref, b1_ref, g1_ref, be1_ref, w2_ref, b2_ref,
              g2_ref, be2_ref, w3_ref, b3_ref, out_ref,
              h1_ref, h2_ref, acc1_ref, acc2_ref):
    p = pl.program_id(0)
    t = pl.program_id(1)

    @pl.when(jnp.logical_and(p == 0, t == 0))
    def _():
        acc1_ref[...] = jnp.zeros_like(acc1_ref)
        acc2_ref[...] = jnp.zeros_like(acc2_ref)

    @pl.when(p == 0)
    def _():
        h = jnp.dot(w1_ref[...], emb_ref[...], preferred_element_type=jnp.float32)
        h = jnp.maximum(h + b1_ref[...], 0.0)
        h1_ref[:, pl.ds(t * BT, BT)] = h
        acc1_ref[:, 0:1] += jnp.sum(h, axis=1, keepdims=True)
        acc1_ref[:, 1:2] += jnp.sum(h * h, axis=1, keepdims=True)

    @pl.when(jnp.logical_and(p == 1, t == 0))
    def _():
        mean = acc1_ref[:, 0:1] * (1.0 / B)
        var = acc1_ref[:, 1:2] * (1.0 / B) - mean * mean
        sc = g1_ref[...] * lax.rsqrt(var + EPS)
        acc1_ref[:, 2:3] = sc
        acc1_ref[:, 3:4] = be1_ref[...] - mean * sc

    @pl.when(p == 1)
    def _():
        h1n = h1_ref[:, pl.ds(t * BT, BT)] * acc1_ref[:, 2:3] + acc1_ref[:, 3:4]
        h = jnp.dot(w2_ref[...], h1n, preferred_element_type=jnp.float32)
        h = jnp.maximum(h + b2_ref[...], 0.0)
        h2_ref[:, pl.ds(t * BT, BT)] = h
        acc2_ref[:, 0:1] += jnp.sum(h, axis=1, keepdims=True)
        acc2_ref[:, 1:2] += jnp.sum(h * h, axis=1, keepdims=True)

    @pl.when(jnp.logical_and(p == 2, t == 0))
    def _():
        mean = acc2_ref[:, 0:1] * (1.0 / B)
        var = acc2_ref[:, 1:2] * (1.0 / B) - mean * mean
        sc = g2_ref[...] * lax.rsqrt(var + EPS)
        acc2_ref[:, 2:3] = sc
        acc2_ref[:, 3:4] = be2_ref[...] - mean * sc

    @pl.when(p == 2)
    def _():
        h2n = h2_ref[:, pl.ds(t * BT, BT)] * acc2_ref[:, 2:3] + acc2_ref[:, 3:4]
        o = jnp.sum(h2n * w3_ref[...], axis=0, keepdims=True) + b3_ref[...]
        out_ref[...] = o


def _mlp_t(embT, W1T, b1, g1, be1, W2T, b2, g2, be2, w3, b3):
    full = lambda shape: pl.BlockSpec(shape, lambda p, t: (0, 0))
    return pl.pallas_call(
        _mlp_body,
        grid=(3, T),
        in_specs=[
            pl.BlockSpec((EM, BT), lambda p, t: (0, jnp.where(p == 0, t, 0))),
            full((H1, EM)), full((H1, 1)), full((H1, 1)), full((H1, 1)),
            full((H2, H1)), full((H2, 1)), full((H2, 1)), full((H2, 1)),
            full((H2, 1)), full((1, 1)),
        ],
        out_specs=pl.BlockSpec((1, BT), lambda p, t: (0, jnp.where(p == 2, t, 0))),
        out_shape=jax.ShapeDtypeStruct((1, B), jnp.float32),
        scratch_shapes=[
            pltpu.VMEM((H1, B), jnp.float32),
            pltpu.VMEM((H2, B), jnp.float32),
            pltpu.VMEM((H1, 8), jnp.float32),
            pltpu.VMEM((H2, 8), jnp.float32),
        ],
        compiler_params=pltpu.CompilerParams(
            dimension_semantics=("arbitrary", "arbitrary")),
    )(embT, W1T, b1, g1, be1, W2T, b2, g2, be2, w3, b3)


def kernel(x, tables, W1, b1, gamma1, beta1, W2, b2, gamma2, beta2, W3, b3):
    xtf = lax.bitcast_convert_type(x.T.astype(jnp.int32), jnp.float32)
    tabT = tables.transpose(0, 2, 1).reshape(EM, V)
    embT = _sc_gather_t(xtf, tabT)
    outT = _mlp_t(embT, W1.T,
                  b1.reshape(H1, 1), gamma1.reshape(H1, 1), beta1.reshape(H1, 1),
                  W2.T, b2.reshape(H2, 1), gamma2.reshape(H2, 1), beta2.reshape(H2, 1),
                  W3, b3.reshape(1, 1))
    return outT[0]
